# Initial kernel scaffold; baseline (speedup 1.0000x reference)
import jax
import jax.numpy as jnp
from jax.experimental import pallas as pl


def kernel(x, pos_Rd, batch_Rd, grid, Bx, BR, W1x, b1x, W2x, b2x, W1r, b1r, W2r, b2r, L1_Wkx, L1_Wkr, L1_W1, L1_b1, L1_W2, L1_b2, L2_Wkx, L2_Wkr, L2_W1, L2_b1, L2_W2, L2_b2):
    raise NotImplementedError("write your pallas kernel here")


# speedup vs baseline: 5.9317x; 5.9317x over previous
import functools

import jax
import jax.numpy as jnp
from jax import lax
from jax.experimental import pallas as pl
from jax.experimental.pallas import tpu as pltpu
from jax.experimental.pallas import tpu_sc as plsc

N = 4096
G = 8
C = 64
BD = 64
K = 8
E = N * K

BRK = 128
BN = 128
TAU = 6.283185307179586


def _knn_body(pos_r, post_r, brow_r, bcol_r, out_r):
    p = pos_r[...]
    pt = post_r[...]
    sqr = jnp.sum(p * p, axis=1, keepdims=True)
    sqc = jnp.sum(pt * pt, axis=0, keepdims=True)
    cross = jnp.dot(p, pt, preferred_element_type=jnp.float32)
    d2 = sqr + sqc - 2.0 * cross
    rb = brow_r[...][:, 0:1]
    cb = bcol_r[...][0:1, :]
    d2 = jnp.where(rb != cb, jnp.inf, d2)
    iota = lax.broadcasted_iota(jnp.int32, d2.shape, 1)
    for k in range(K):
        rowmin = jnp.min(d2, axis=1, keepdims=True)
        cand = jnp.where(d2 <= rowmin, iota, N)
        idx = jnp.min(cand, axis=1, keepdims=True)
        out_r[:, k:k + 1] = idx
        d2 = jnp.where(iota == idx, jnp.inf, d2)


def _knn_call(posp8, post8, brow, bcol):
    return pl.pallas_call(
        _knn_body,
        grid=(N // BRK,),
        in_specs=[
            pl.BlockSpec((BRK, 8), lambda b: (b, 0)),
            pl.BlockSpec((8, N), lambda b: (0, 0)),
            pl.BlockSpec((BRK, 8), lambda b: (b, 0)),
            pl.BlockSpec((8, N), lambda b: (0, 0)),
        ],
        out_specs=pl.BlockSpec((BRK, K), lambda b: (b, 0)),
        out_shape=jax.ShapeDtypeStruct((N, K), jnp.int32),
    )(posp8, post8, brow, bcol)


_SC_CH = 128
_SC_NW = 32
_SC_ROWS = E // _SC_NW
_SC_NCH = _SC_ROWS // _SC_CH


def _gather_edges(table_x, table_p, idx):
    mesh = plsc.VectorSubcoreMesh(core_axis_name="c", subcore_axis_name="s")

    @functools.partial(
        pl.kernel,
        mesh=mesh,
        out_type=(
            jax.ShapeDtypeStruct((E, 512), jnp.float32),
            jax.ShapeDtypeStruct((E, 128), jnp.float32),
        ),
        scratch_types=[
            pltpu.VMEM((_SC_CH,), jnp.int32),
            pltpu.VMEM((_SC_CH, 512), jnp.float32),
            pltpu.VMEM((_SC_CH, 128), jnp.float32),
            pltpu.SemaphoreType.DMA,
            pltpu.SemaphoreType.DMA,
        ],
    )
    def gk(tx_hbm, tp_hbm, idx_hbm, ox_hbm, op_hbm, idx_v, rx_v, rp_v, s1, s2):
        wid = lax.axis_index("s") * 2 + lax.axis_index("c")
        base = wid * _SC_ROWS

        def chunk(i, carry):
            off = base + i * _SC_CH
            pltpu.sync_copy(idx_hbm.at[pl.ds(off, _SC_CH)], idx_v)
            a = pltpu.async_copy(tx_hbm.at[idx_v], rx_v, s1)
            b = pltpu.async_copy(tp_hbm.at[idx_v], rp_v, s2)
            a.wait()
            b.wait()
            pltpu.sync_copy(rx_v, ox_hbm.at[pl.ds(off, _SC_CH)])
            pltpu.sync_copy(rp_v, op_hbm.at[pl.ds(off, _SC_CH)])
            return carry

        lax.fori_loop(0, _SC_NCH, chunk, 0)

    return gk(table_x, table_p, idx)


def _layer_body(g_r, gp_r, pos_r, x_r, gt_r, grd_r, relR_r, bx_r, br_r,
                w1x_r, b1x_r, w2x_r, b2x_r, w1r_r, b1r_r, w2r_r, b2r_r,
                wkx_r, wkr_r, w1_r, b1_r, w2_r, b2_r, out_r):
    Er = K * BN
    gp = gp_r[...].reshape(Er, 128)
    pos = pos_r[...]
    posrep = jnp.concatenate([pos] * K, axis=0)
    rel = gp - posrep
    rz = jnp.dot(rel, gt_r[...], preferred_element_type=jnp.float32)
    grd = grd_r[...]

    bx0 = bx_r[...][0:1, :]
    bx1 = bx_r[...][1:2, :]
    w1x = w1x_r[...]
    b1x = b1x_r[...]
    w2x = w2x_r[...]
    b2x = b2x_r[...]

    g2 = g_r[...].reshape(Er, 512)
    x = x_r[...]

    pR = TAU * (relR_r[...] * br_r[...])
    featR = jnp.concatenate([jnp.sin(pR), jnp.cos(pR)], axis=1)
    hR = jax.nn.gelu(jnp.dot(featR, w1r_r[...],
                             preferred_element_type=jnp.float32) + b1r_r[...])
    kbR = jax.nn.gelu(jnp.dot(hR, w2r_r[...],
                              preferred_element_type=jnp.float32) + b2r_r[...])
    kR = jnp.dot(kbR, wkr_r[...], preferred_element_type=jnp.float32)

    aggs = []
    for m in range(G):
        rzm = rz[:, m:m + 1]
        diff = rel - rzm * grd[m:m + 1, :]
        rxym = jnp.sqrt(jnp.sum(diff * diff, axis=1, keepdims=True) + 1e-12)
        pm = TAU * (rxym * bx0 + rzm * bx1)
        feat = jnp.concatenate([jnp.sin(pm), jnp.cos(pm)], axis=1)
        h = jax.nn.gelu(jnp.dot(feat, w1x,
                                preferred_element_type=jnp.float32) + b1x)
        kb = jax.nn.gelu(jnp.dot(h, w2x,
                                 preferred_element_type=jnp.float32) + b2x)
        kx = jnp.dot(kb, wkx_r[...], preferred_element_type=jnp.float32)
        msg = g2[:, m * C:(m + 1) * C] * kx
        agg = msg[0:BN]
        for k in range(1, K):
            agg = agg + msg[k * BN:(k + 1) * BN]
        aggs.append(agg * 0.125)

    for gi in range(G):
        xg = aggs[0] * kR[gi * G:gi * G + 1, :]
        for m in range(1, G):
            xg = xg + aggs[m] * kR[gi * G + m:gi * G + m + 1, :]
        xg = xg * 0.125
        h = jax.nn.gelu(jnp.dot(xg, w1_r[...],
                                preferred_element_type=jnp.float32) + b1_r[...])
        h = jnp.dot(h, w2_r[...], preferred_element_type=jnp.float32) + b2_r[...]
        out_r[:, gi * C:(gi + 1) * C] = x[:, gi * C:(gi + 1) * C] + h


def _layer_specs():
    full = lambda shape: pl.BlockSpec(shape, lambda b: tuple(0 for _ in shape))
    in_specs = [
        pl.BlockSpec((K, BN, 512), lambda b: (0, b, 0)),
        pl.BlockSpec((K, BN, 128), lambda b: (0, b, 0)),
        pl.BlockSpec((BN, 128), lambda b: (b, 0)),
        pl.BlockSpec((BN, 512), lambda b: (b, 0)),
        full((128, G)), full((G, 128)), full((G * G, 1)),
        full((2, C // 2)), full((1, C // 2)),
        full((C, C)), full((1, C)), full((C, BD)), full((1, BD)),
        full((C, C)), full((1, C)), full((C, BD)), full((1, BD)),
        full((BD, C)), full((BD, C)),
        full((C, C)), full((1, C)), full((C, C)), full((1, C)),
    ]
    out_spec = pl.BlockSpec((BN, 512), lambda b: (b, 0))
    return in_specs, out_spec


def _layer_call(args):
    in_specs, out_spec = _layer_specs()
    return pl.pallas_call(
        _layer_body,
        grid=(N // BN,),
        in_specs=in_specs,
        out_specs=out_spec,
        out_shape=jax.ShapeDtypeStruct((N, 512), jnp.float32),
    )(*args)


def kernel(x, pos_Rd, batch_Rd, grid, Bx, BR, W1x, b1x, W2x, b2x,
           W1r, b1r, W2r, b2r, L1_Wkx, L1_Wkr, L1_W1, L1_b1, L1_W2, L1_b2,
           L2_Wkx, L2_Wkr, L2_W1, L2_b1, L2_W2, L2_b2):
    x2d = x.reshape(N, G * C)
    posp8 = jnp.pad(pos_Rd, ((0, 0), (0, 5)))
    post8 = posp8.T
    posp128 = jnp.pad(pos_Rd, ((0, 0), (0, 125)))
    bi = batch_Rd.astype(jnp.int32)
    brow = jnp.broadcast_to(bi[:, None], (N, 8))
    bcol = jnp.broadcast_to(bi[None, :], (8, N))

    nbr = _knn_call(posp8, post8, brow, bcol)
    idx = nbr.T.reshape(E)

    grid128 = jnp.pad(grid, ((0, 0), (0, 125)))
    gridT128 = grid128.T
    relR = (grid @ grid.T).reshape(G * G, 1)

    g1, gp1 = _gather_edges(x2d, posp128, idx)
    g13 = g1.reshape(K, N, 512)
    gp13 = gp1.reshape(K, N, 128)

    def weights(Wkx, Wkr, W1, b1, W2, b2):
        return (gridT128, grid128, relR, Bx, BR.reshape(1, C // 2),
                W1x, b1x.reshape(1, C), W2x, b2x.reshape(1, BD),
                W1r, b1r.reshape(1, C), W2r, b2r.reshape(1, BD),
                Wkx, Wkr, W1, b1.reshape(1, C), W2, b2.reshape(1, C))

    x1 = _layer_call((g13, gp13, posp128, x2d)
                     + weights(L1_Wkx, L1_Wkr, L1_W1, L1_b1, L1_W2, L1_b2))

    g2, _ = _gather_edges(x1, posp128, idx)
    x2 = _layer_call((g2.reshape(K, N, 512), gp13, posp128, x1)
                     + weights(L2_Wkx, L2_Wkr, L2_W1, L2_b1, L2_W2, L2_b2))

    return x2.reshape(N, G, C)


# speedup vs baseline: 9.3480x; 1.5760x over previous
import functools

import jax
import jax.numpy as jnp
from jax import lax
from jax.experimental import pallas as pl
from jax.experimental.pallas import tpu as pltpu
from jax.experimental.pallas import tpu_sc as plsc

N = 4096
G = 8
C = 64
BD = 64
K = 8
E = N * K

BRK = 128
BN = 128
TAU = 6.283185307179586


def _knn_body(pos_r, post_r, brow_r, bcol_r, out_r):
    p = pos_r[...]
    pt = post_r[...]
    sqr = jnp.sum(p * p, axis=1, keepdims=True)
    sqc = jnp.sum(pt * pt, axis=0, keepdims=True)
    cross = jnp.dot(p, pt, preferred_element_type=jnp.float32)
    d2 = sqr + sqc - 2.0 * cross
    rb = brow_r[...][:, 0:1]
    cb = bcol_r[...][0:1, :]
    d2 = jnp.where(rb != cb, jnp.inf, d2)
    iota = lax.broadcasted_iota(jnp.int32, d2.shape, 1)
    for k in range(K):
        rowmin = jnp.min(d2, axis=1, keepdims=True)
        cand = jnp.where(d2 <= rowmin, iota, N)
        idx = jnp.min(cand, axis=1, keepdims=True)
        out_r[:, k:k + 1] = idx
        d2 = jnp.where(iota == idx, jnp.inf, d2)


def _knn_call(posp8, post8, brow, bcol):
    return pl.pallas_call(
        _knn_body,
        grid=(N // BRK,),
        in_specs=[
            pl.BlockSpec((BRK, 8), lambda b: (b, 0)),
            pl.BlockSpec((8, N), lambda b: (0, 0)),
            pl.BlockSpec((BRK, 8), lambda b: (b, 0)),
            pl.BlockSpec((8, N), lambda b: (0, 0)),
        ],
        out_specs=pl.BlockSpec((BRK, K), lambda b: (b, 0)),
        out_shape=jax.ShapeDtypeStruct((N, K), jnp.int32),
    )(posp8, post8, brow, bcol)


_SC_CH = 128
_SC_NW = 32
_SC_ROWS = E // _SC_NW
_SC_NCH = _SC_ROWS // _SC_CH


def _gather_edges(table_x, table_p, idx):
    mesh = plsc.VectorSubcoreMesh(core_axis_name="c", subcore_axis_name="s")

    @functools.partial(
        pl.kernel,
        mesh=mesh,
        out_type=(
            jax.ShapeDtypeStruct((E, 512), jnp.float32),
            jax.ShapeDtypeStruct((E, 128), jnp.float32),
        ),
        scratch_types=[
            pltpu.VMEM((_SC_CH,), jnp.int32),
            pltpu.VMEM((_SC_CH, 512), jnp.float32),
            pltpu.VMEM((_SC_CH, 128), jnp.float32),
            pltpu.SemaphoreType.DMA,
            pltpu.SemaphoreType.DMA,
        ],
    )
    def gk(tx_hbm, tp_hbm, idx_hbm, ox_hbm, op_hbm, idx_v, rx_v, rp_v, s1, s2):
        wid = lax.axis_index("s") * 2 + lax.axis_index("c")
        base = wid * _SC_ROWS

        def chunk(i, carry):
            off = base + i * _SC_CH
            pltpu.sync_copy(idx_hbm.at[pl.ds(off, _SC_CH)], idx_v)
            a = pltpu.async_copy(tx_hbm.at[idx_v], rx_v, s1)
            b = pltpu.async_copy(tp_hbm.at[idx_v], rp_v, s2)
            a.wait()
            b.wait()
            pltpu.sync_copy(rx_v, ox_hbm.at[pl.ds(off, _SC_CH)])
            pltpu.sync_copy(rp_v, op_hbm.at[pl.ds(off, _SC_CH)])
            return carry

        lax.fori_loop(0, _SC_NCH, chunk, 0)

    return gk(table_x, table_p, idx)


def _gather_edges_x(table_x, idx):
    mesh = plsc.VectorSubcoreMesh(core_axis_name="c", subcore_axis_name="s")

    @functools.partial(
        pl.kernel,
        mesh=mesh,
        out_type=jax.ShapeDtypeStruct((E, 512), jnp.float32),
        scratch_types=[
            pltpu.VMEM((_SC_CH,), jnp.int32),
            pltpu.VMEM((_SC_CH, 512), jnp.float32),
            pltpu.SemaphoreType.DMA,
        ],
    )
    def gk(tx_hbm, idx_hbm, ox_hbm, idx_v, rx_v, s1):
        wid = lax.axis_index("s") * 2 + lax.axis_index("c")
        base = wid * _SC_ROWS

        def chunk(i, carry):
            off = base + i * _SC_CH
            pltpu.sync_copy(idx_hbm.at[pl.ds(off, _SC_CH)], idx_v)
            pltpu.async_copy(tx_hbm.at[idx_v], rx_v, s1).wait()
            pltpu.sync_copy(rx_v, ox_hbm.at[pl.ds(off, _SC_CH)])
            return carry

        lax.fori_loop(0, _SC_NCH, chunk, 0)

    return gk(table_x, idx)


def _kR_mix(relR_r, br_r, w1r_r, b1r_r, w2r_r, b2r_r, wkr_r):
    pR = TAU * (relR_r[...] * br_r[...])
    featR = jnp.concatenate([jnp.sin(pR), jnp.cos(pR)], axis=1)
    hR = jax.nn.gelu(jnp.dot(featR, w1r_r[...],
                             preferred_element_type=jnp.float32) + b1r_r[...])
    kbR = jax.nn.gelu(jnp.dot(hR, w2r_r[...],
                              preferred_element_type=jnp.float32) + b2r_r[...])
    return jnp.dot(kbR, wkr_r[...], preferred_element_type=jnp.float32)


def _node_update(aggs, kR, x, w1_r, b1_r, w2_r, b2_r, out_r):
    for gi in range(G):
        xg = aggs[0] * kR[gi * G:gi * G + 1, :]
        for m in range(1, G):
            xg = xg + aggs[m] * kR[gi * G + m:gi * G + m + 1, :]
        xg = xg * 0.125
        h = jax.nn.gelu(jnp.dot(xg, w1_r[...],
                                preferred_element_type=jnp.float32) + b1_r[...])
        h = jnp.dot(h, w2_r[...], preferred_element_type=jnp.float32) + b2_r[...]
        out_r[:, gi * C:(gi + 1) * C] = x[:, gi * C:(gi + 1) * C] + h


def _layer1_body(g_r, gp_r, pos_r, x_r, gt_r, grd_r, relR_r, bx_r, br_r,
                 w1x_r, b1x_r, w2x_r, b2x_r, w1r_r, b1r_r, w2r_r, b2r_r,
                 wkx_r, wkx2_r, wkr_r, w1_r, b1_r, w2_r, b2_r, out_r, out2_r):
    Er = K * BN
    gp = gp_r[...].reshape(Er, 128)
    pos = pos_r[...]
    posrep = jnp.concatenate([pos] * K, axis=0)
    rel = gp - posrep
    rz = jnp.dot(rel, gt_r[...], preferred_element_type=jnp.float32)
    grd = grd_r[...]

    bx0 = bx_r[...][0:1, :]
    bx1 = bx_r[...][1:2, :]
    w1x = w1x_r[...]
    b1x = b1x_r[...]
    w2x = w2x_r[...]
    b2x = b2x_r[...]

    g2 = g_r[...].reshape(Er, 512)
    x = x_r[...]

    kR = _kR_mix(relR_r, br_r, w1r_r, b1r_r, w2r_r, b2r_r, wkr_r)

    aggs = []
    for m in range(G):
        rzm = rz[:, m:m + 1]
        diff = rel - rzm * grd[m:m + 1, :]
        rxym = jnp.sqrt(jnp.sum(diff * diff, axis=1, keepdims=True) + 1e-12)
        pm = TAU * (rxym * bx0 + rzm * bx1)
        feat = jnp.concatenate([jnp.sin(pm), jnp.cos(pm)], axis=1)
        h = jax.nn.gelu(jnp.dot(feat, w1x,
                                preferred_element_type=jnp.float32) + b1x)
        kb = jax.nn.gelu(jnp.dot(h, w2x,
                                 preferred_element_type=jnp.float32) + b2x)
        kx = jnp.dot(kb, wkx_r[...], preferred_element_type=jnp.float32)
        kx2 = jnp.dot(kb, wkx2_r[...], preferred_element_type=jnp.float32)
        out2_r[:, :, m * C:(m + 1) * C] = kx2.reshape(K, BN, C)
        msg = g2[:, m * C:(m + 1) * C] * kx
        agg = msg[0:BN]
        for k in range(1, K):
            agg = agg + msg[k * BN:(k + 1) * BN]
        aggs.append(agg * 0.125)

    _node_update(aggs, kR, x, w1_r, b1_r, w2_r, b2_r, out_r)


def _layer2_body(g_r, kx2_r, x_r, relR_r, br_r,
                 w1r_r, b1r_r, w2r_r, b2r_r, wkr_r,
                 w1_r, b1_r, w2_r, b2_r, out_r):
    Er = K * BN
    g2 = g_r[...].reshape(Er, 512)
    kxa = kx2_r[...].reshape(Er, 512)
    x = x_r[...]

    kR = _kR_mix(relR_r, br_r, w1r_r, b1r_r, w2r_r, b2r_r, wkr_r)

    msg = g2 * kxa
    acc = msg[0:BN]
    for k in range(1, K):
        acc = acc + msg[k * BN:(k + 1) * BN]
    acc = acc * 0.125
    aggs = [acc[:, m * C:(m + 1) * C] for m in range(G)]

    _node_update(aggs, kR, x, w1_r, b1_r, w2_r, b2_r, out_r)


def _layer1_specs():
    full = lambda shape: pl.BlockSpec(shape, lambda b: tuple(0 for _ in shape))
    in_specs = [
        pl.BlockSpec((K, BN, 512), lambda b: (0, b, 0)),
        pl.BlockSpec((K, BN, 128), lambda b: (0, b, 0)),
        pl.BlockSpec((BN, 128), lambda b: (b, 0)),
        pl.BlockSpec((BN, 512), lambda b: (b, 0)),
        full((128, G)), full((G, 128)), full((G * G, 1)),
        full((2, C // 2)), full((1, C // 2)),
        full((C, C)), full((1, C)), full((C, BD)), full((1, BD)),
        full((C, C)), full((1, C)), full((C, BD)), full((1, BD)),
        full((BD, C)), full((BD, C)), full((BD, C)),
        full((C, C)), full((1, C)), full((C, C)), full((1, C)),
    ]
    out_specs = [
        pl.BlockSpec((BN, 512), lambda b: (b, 0)),
        pl.BlockSpec((K, BN, 512), lambda b: (0, b, 0)),
    ]
    return in_specs, out_specs


def _layer1_call(args):
    in_specs, out_specs = _layer1_specs()
    return pl.pallas_call(
        _layer1_body,
        grid=(N // BN,),
        in_specs=in_specs,
        out_specs=out_specs,
        out_shape=[
            jax.ShapeDtypeStruct((N, 512), jnp.float32),
            jax.ShapeDtypeStruct((K, N, 512), jnp.float32),
        ],
    )(*args)


def _layer2_specs():
    full = lambda shape: pl.BlockSpec(shape, lambda b: tuple(0 for _ in shape))
    in_specs = [
        pl.BlockSpec((K, BN, 512), lambda b: (0, b, 0)),
        pl.BlockSpec((K, BN, 512), lambda b: (0, b, 0)),
        pl.BlockSpec((BN, 512), lambda b: (b, 0)),
        full((G * G, 1)), full((1, C // 2)),
        full((C, C)), full((1, C)), full((C, BD)), full((1, BD)),
        full((BD, C)),
        full((C, C)), full((1, C)), full((C, C)), full((1, C)),
    ]
    out_spec = pl.BlockSpec((BN, 512), lambda b: (b, 0))
    return in_specs, out_spec


def _layer2_call(args):
    in_specs, out_spec = _layer2_specs()
    return pl.pallas_call(
        _layer2_body,
        grid=(N // BN,),
        in_specs=in_specs,
        out_specs=out_spec,
        out_shape=jax.ShapeDtypeStruct((N, 512), jnp.float32),
    )(*args)


def kernel(x, pos_Rd, batch_Rd, grid, Bx, BR, W1x, b1x, W2x, b2x,
           W1r, b1r, W2r, b2r, L1_Wkx, L1_Wkr, L1_W1, L1_b1, L1_W2, L1_b2,
           L2_Wkx, L2_Wkr, L2_W1, L2_b1, L2_W2, L2_b2):
    x2d = x.reshape(N, G * C)
    posp8 = jnp.pad(pos_Rd, ((0, 0), (0, 5)))
    post8 = posp8.T
    posp128 = jnp.pad(pos_Rd, ((0, 0), (0, 125)))
    bi = batch_Rd.astype(jnp.int32)
    brow = jnp.broadcast_to(bi[:, None], (N, 8))
    bcol = jnp.broadcast_to(bi[None, :], (8, N))

    nbr = _knn_call(posp8, post8, brow, bcol)
    idx = nbr.T.reshape(E)

    grid128 = jnp.pad(grid, ((0, 0), (0, 125)))
    gridT128 = grid128.T
    relR = (grid @ grid.T).reshape(G * G, 1)

    g1, gp1 = _gather_edges(x2d, posp128, idx)
    g13 = g1.reshape(K, N, 512)
    gp13 = gp1.reshape(K, N, 128)

    x1, kx2 = _layer1_call(
        (g13, gp13, posp128, x2d, gridT128, grid128, relR,
         Bx, BR.reshape(1, C // 2),
         W1x, b1x.reshape(1, C), W2x, b2x.reshape(1, BD),
         W1r, b1r.reshape(1, C), W2r, b2r.reshape(1, BD),
         L1_Wkx, L2_Wkx, L1_Wkr,
         L1_W1, L1_b1.reshape(1, C), L1_W2, L1_b2.reshape(1, C)))

    g2 = _gather_edges_x(x1, idx)
    x2 = _layer2_call(
        (g2.reshape(K, N, 512), kx2, x1, relR, BR.reshape(1, C // 2),
         W1r, b1r.reshape(1, C), W2r, b2r.reshape(1, BD),
         L2_Wkr,
         L2_W1, L2_b1.reshape(1, C), L2_W2, L2_b2.reshape(1, C)))

    return x2.reshape(N, G, C)


# speedup vs baseline: 12.4490x; 1.3317x over previous
import functools

import jax
import jax.numpy as jnp
from jax import lax
from jax.experimental import pallas as pl
from jax.experimental.pallas import tpu as pltpu
from jax.experimental.pallas import tpu_sc as plsc

N = 4096
G = 8
C = 64
BD = 64
K = 8
E = N * K

BRK = 128
BN = 128
TAU = 6.283185307179586


def _knn_body(pos_r, post_r, brow_r, bcol_r, out_r):
    p = pos_r[...]
    pt = post_r[...]
    sqr = jnp.sum(p * p, axis=1, keepdims=True)
    sqc = jnp.sum(pt * pt, axis=0, keepdims=True)
    cross = jnp.dot(p, pt, preferred_element_type=jnp.float32)
    d2 = sqr + sqc - 2.0 * cross
    rb = brow_r[...][:, 0:1]
    cb = bcol_r[...][0:1, :]
    d2 = jnp.where(rb != cb, jnp.inf, d2)
    iota = lax.broadcasted_iota(jnp.int32, d2.shape, 1)
    for k in range(K):
        rowmin = jnp.min(d2, axis=1, keepdims=True)
        cand = jnp.where(d2 <= rowmin, iota, N)
        idx = jnp.min(cand, axis=1, keepdims=True)
        out_r[:, k:k + 1] = idx
        d2 = jnp.where(iota == idx, jnp.inf, d2)


def _knn_call(posp8, post8, brow, bcol):
    return pl.pallas_call(
        _knn_body,
        grid=(N // BRK,),
        in_specs=[
            pl.BlockSpec((BRK, 8), lambda b: (b, 0)),
            pl.BlockSpec((8, N), lambda b: (0, 0)),
            pl.BlockSpec((BRK, 8), lambda b: (b, 0)),
            pl.BlockSpec((8, N), lambda b: (0, 0)),
        ],
        out_specs=pl.BlockSpec((BRK, K), lambda b: (b, 0)),
        out_shape=jax.ShapeDtypeStruct((N, K), jnp.int32),
    )(posp8, post8, brow, bcol)


_SC_CH = 128
_SC_NW = 32
_SC_ROWS = E // _SC_NW
_SC_NCH = _SC_ROWS // _SC_CH


def _gather_edges(table_x, table_p, idx):
    mesh = plsc.VectorSubcoreMesh(core_axis_name="c", subcore_axis_name="s")

    @functools.partial(
        pl.kernel,
        mesh=mesh,
        out_type=(
            jax.ShapeDtypeStruct((E, 512), jnp.float32),
            jax.ShapeDtypeStruct((E, 128), jnp.float32),
        ),
        scratch_types=[
            pltpu.VMEM((_SC_CH,), jnp.int32),
            pltpu.VMEM((_SC_CH, 512), jnp.float32),
            pltpu.VMEM((_SC_CH, 128), jnp.float32),
            pltpu.SemaphoreType.DMA,
            pltpu.SemaphoreType.DMA,
        ],
    )
    def gk(tx_hbm, tp_hbm, idx_hbm, ox_hbm, op_hbm, idx_v, rx_v, rp_v, s1, s2):
        wid = lax.axis_index("s") * 2 + lax.axis_index("c")
        base = wid * _SC_ROWS

        def chunk(i, carry):
            off = base + i * _SC_CH
            pltpu.sync_copy(idx_hbm.at[pl.ds(off, _SC_CH)], idx_v)
            a = pltpu.async_copy(tx_hbm.at[idx_v], rx_v, s1)
            b = pltpu.async_copy(tp_hbm.at[idx_v], rp_v, s2)
            a.wait()
            b.wait()
            pltpu.sync_copy(rx_v, ox_hbm.at[pl.ds(off, _SC_CH)])
            pltpu.sync_copy(rp_v, op_hbm.at[pl.ds(off, _SC_CH)])
            return carry

        lax.fori_loop(0, _SC_NCH, chunk, 0)

    return gk(table_x, table_p, idx)


def _gather_edges_x(table_x, idx):
    mesh = plsc.VectorSubcoreMesh(core_axis_name="c", subcore_axis_name="s")

    @functools.partial(
        pl.kernel,
        mesh=mesh,
        out_type=jax.ShapeDtypeStruct((E, 512), jnp.float32),
        scratch_types=[
            pltpu.VMEM((_SC_CH,), jnp.int32),
            pltpu.VMEM((_SC_CH, 512), jnp.float32),
            pltpu.SemaphoreType.DMA,
        ],
    )
    def gk(tx_hbm, idx_hbm, ox_hbm, idx_v, rx_v, s1):
        wid = lax.axis_index("s") * 2 + lax.axis_index("c")
        base = wid * _SC_ROWS

        def chunk(i, carry):
            off = base + i * _SC_CH
            pltpu.sync_copy(idx_hbm.at[pl.ds(off, _SC_CH)], idx_v)
            pltpu.async_copy(tx_hbm.at[idx_v], rx_v, s1).wait()
            pltpu.sync_copy(rx_v, ox_hbm.at[pl.ds(off, _SC_CH)])
            return carry

        lax.fori_loop(0, _SC_NCH, chunk, 0)

    return gk(table_x, idx)


_SS = (6.283185005187988, -41.3416633605957, 81.60163116455078,
       -76.56468200683594, 39.652915954589844)
_CC = (1.0, -19.739208221435547, 64.9393539428711, -85.45401000976562,
       60.15278244018555, -25.04315948486328)


def _sincos_2pi(q):
    n2 = jnp.round(2.0 * q)
    v = q - 0.5 * n2
    odd = lax.convert_element_type(n2, jnp.int32) << 31
    u = v * v
    s = _SS[4]
    for k in (3, 2, 1, 0):
        s = u * s + _SS[k]
    s = v * s
    c = _CC[5]
    for k in (4, 3, 2, 1, 0):
        c = u * c + _CC[k]
    sb = lax.bitcast_convert_type(s, jnp.int32) ^ odd
    cb = lax.bitcast_convert_type(c, jnp.int32) ^ odd
    return (lax.bitcast_convert_type(sb, jnp.float32),
            lax.bitcast_convert_type(cb, jnp.float32))


def _kR_mix(relR_r, br_r, w1r_r, b1r_r, w2r_r, b2r_r, wkr_r):
    pR = TAU * (relR_r[...] * br_r[...])
    featR = jnp.concatenate([jnp.sin(pR), jnp.cos(pR)], axis=1)
    hR = jax.nn.gelu(jnp.dot(featR, w1r_r[...],
                             preferred_element_type=jnp.float32) + b1r_r[...])
    kbR = jax.nn.gelu(jnp.dot(hR, w2r_r[...],
                              preferred_element_type=jnp.float32) + b2r_r[...])
    return jnp.dot(kbR, wkr_r[...], preferred_element_type=jnp.float32)


def _node_update(aggs, kR, x, w1_r, b1_r, w2_r, b2_r, out_r):
    for gi in range(G):
        xg = aggs[0] * kR[gi * G:gi * G + 1, :]
        for m in range(1, G):
            xg = xg + aggs[m] * kR[gi * G + m:gi * G + m + 1, :]
        xg = xg * 0.125
        h = jax.nn.gelu(jnp.dot(xg, w1_r[...],
                                preferred_element_type=jnp.float32) + b1_r[...])
        h = jnp.dot(h, w2_r[...], preferred_element_type=jnp.float32) + b2_r[...]
        out_r[:, gi * C:(gi + 1) * C] = x[:, gi * C:(gi + 1) * C] + h


def _layer1_body(g_r, gp_r, pos_r, x_r, gt_r, grd_r, relR_r, bx_r, br_r,
                 w1x_r, b1x_r, w2x_r, b2x_r, w1r_r, b1r_r, w2r_r, b2r_r,
                 wkx_r, wkx2_r, wkr_r, w1_r, b1_r, w2_r, b2_r, out_r, out2_r):
    Er = K * BN
    gp = gp_r[...].reshape(Er, 128)
    pos = pos_r[...]
    posrep = jnp.concatenate([pos] * K, axis=0)
    rel = gp - posrep
    rz = jnp.dot(rel, gt_r[...], preferred_element_type=jnp.float32)
    grd = grd_r[...]

    bx0 = bx_r[...][0:1, :]
    bx1 = bx_r[...][1:2, :]
    w1x = w1x_r[...]
    b1x = b1x_r[...]
    w2x = w2x_r[...]
    b2x = b2x_r[...]

    g2 = g_r[...].reshape(Er, 512)
    x = x_r[...]

    kR = _kR_mix(relR_r, br_r, w1r_r, b1r_r, w2r_r, b2r_r, wkr_r)

    aggs = []
    for m in range(G):
        rzm = rz[:, m:m + 1]
        diff = rel - rzm * grd[m:m + 1, :]
        rxym = jnp.sqrt(jnp.sum(diff * diff, axis=1, keepdims=True) + 1e-12)
        qm = rxym * bx0 + rzm * bx1
        sm, cm = _sincos_2pi(qm)
        feat = jnp.concatenate([sm, cm], axis=1)
        h = jax.nn.gelu(jnp.dot(feat, w1x,
                                preferred_element_type=jnp.float32) + b1x)
        kb = jax.nn.gelu(jnp.dot(h, w2x,
                                 preferred_element_type=jnp.float32) + b2x)
        kx = jnp.dot(kb, wkx_r[...], preferred_element_type=jnp.float32)
        kx2 = jnp.dot(kb, wkx2_r[...], preferred_element_type=jnp.float32)
        out2_r[:, :, m * C:(m + 1) * C] = kx2.reshape(K, BN, C)
        msg = g2[:, m * C:(m + 1) * C] * kx
        agg = msg[0:BN]
        for k in range(1, K):
            agg = agg + msg[k * BN:(k + 1) * BN]
        aggs.append(agg * 0.125)

    _node_update(aggs, kR, x, w1_r, b1_r, w2_r, b2_r, out_r)


def _layer2_body(g_r, kx2_r, x_r, relR_r, br_r,
                 w1r_r, b1r_r, w2r_r, b2r_r, wkr_r,
                 w1_r, b1_r, w2_r, b2_r, out_r):
    Er = K * BN
    g2 = g_r[...].reshape(Er, 512)
    kxa = kx2_r[...].reshape(Er, 512)
    x = x_r[...]

    kR = _kR_mix(relR_r, br_r, w1r_r, b1r_r, w2r_r, b2r_r, wkr_r)

    msg = g2 * kxa
    acc = msg[0:BN]
    for k in range(1, K):
        acc = acc + msg[k * BN:(k + 1) * BN]
    acc = acc * 0.125
    aggs = [acc[:, m * C:(m + 1) * C] for m in range(G)]

    _node_update(aggs, kR, x, w1_r, b1_r, w2_r, b2_r, out_r)


def _layer1_specs():
    full = lambda shape: pl.BlockSpec(shape, lambda b: tuple(0 for _ in shape))
    in_specs = [
        pl.BlockSpec((K, BN, 512), lambda b: (0, b, 0)),
        pl.BlockSpec((K, BN, 128), lambda b: (0, b, 0)),
        pl.BlockSpec((BN, 128), lambda b: (b, 0)),
        pl.BlockSpec((BN, 512), lambda b: (b, 0)),
        full((128, G)), full((G, 128)), full((G * G, 1)),
        full((2, C // 2)), full((1, C // 2)),
        full((C, C)), full((1, C)), full((C, BD)), full((1, BD)),
        full((C, C)), full((1, C)), full((C, BD)), full((1, BD)),
        full((BD, C)), full((BD, C)), full((BD, C)),
        full((C, C)), full((1, C)), full((C, C)), full((1, C)),
    ]
    out_specs = [
        pl.BlockSpec((BN, 512), lambda b: (b, 0)),
        pl.BlockSpec((K, BN, 512), lambda b: (0, b, 0)),
    ]
    return in_specs, out_specs


def _layer1_call(args):
    in_specs, out_specs = _layer1_specs()
    return pl.pallas_call(
        _layer1_body,
        grid=(N // BN,),
        in_specs=in_specs,
        out_specs=out_specs,
        out_shape=[
            jax.ShapeDtypeStruct((N, 512), jnp.float32),
            jax.ShapeDtypeStruct((K, N, 512), jnp.float32),
        ],
    )(*args)


def _layer2_specs():
    full = lambda shape: pl.BlockSpec(shape, lambda b: tuple(0 for _ in shape))
    in_specs = [
        pl.BlockSpec((K, BN, 512), lambda b: (0, b, 0)),
        pl.BlockSpec((K, BN, 512), lambda b: (0, b, 0)),
        pl.BlockSpec((BN, 512), lambda b: (b, 0)),
        full((G * G, 1)), full((1, C // 2)),
        full((C, C)), full((1, C)), full((C, BD)), full((1, BD)),
        full((BD, C)),
        full((C, C)), full((1, C)), full((C, C)), full((1, C)),
    ]
    out_spec = pl.BlockSpec((BN, 512), lambda b: (b, 0))
    return in_specs, out_spec


def _layer2_call(args):
    in_specs, out_spec = _layer2_specs()
    return pl.pallas_call(
        _layer2_body,
        grid=(N // BN,),
        in_specs=in_specs,
        out_specs=out_spec,
        out_shape=jax.ShapeDtypeStruct((N, 512), jnp.float32),
    )(*args)


def kernel(x, pos_Rd, batch_Rd, grid, Bx, BR, W1x, b1x, W2x, b2x,
           W1r, b1r, W2r, b2r, L1_Wkx, L1_Wkr, L1_W1, L1_b1, L1_W2, L1_b2,
           L2_Wkx, L2_Wkr, L2_W1, L2_b1, L2_W2, L2_b2):
    x2d = x.reshape(N, G * C)
    posp8 = jnp.pad(pos_Rd, ((0, 0), (0, 5)))
    post8 = posp8.T
    posp128 = jnp.pad(pos_Rd, ((0, 0), (0, 125)))
    bi = batch_Rd.astype(jnp.int32)
    brow = jnp.broadcast_to(bi[:, None], (N, 8))
    bcol = jnp.broadcast_to(bi[None, :], (8, N))

    nbr = _knn_call(posp8, post8, brow, bcol)
    idx = nbr.T.reshape(E)

    grid128 = jnp.pad(grid, ((0, 0), (0, 125)))
    gridT128 = grid128.T
    relR = (grid @ grid.T).reshape(G * G, 1)

    g1, gp1 = _gather_edges(x2d, posp128, idx)
    g13 = g1.reshape(K, N, 512)
    gp13 = gp1.reshape(K, N, 128)

    x1, kx2 = _layer1_call(
        (g13, gp13, posp128, x2d, gridT128, grid128, relR,
         Bx, BR.reshape(1, C // 2),
         W1x, b1x.reshape(1, C), W2x, b2x.reshape(1, BD),
         W1r, b1r.reshape(1, C), W2r, b2r.reshape(1, BD),
         L1_Wkx, L2_Wkx, L1_Wkr,
         L1_W1, L1_b1.reshape(1, C), L1_W2, L1_b2.reshape(1, C)))

    g2 = _gather_edges_x(x1, idx)
    x2 = _layer2_call(
        (g2.reshape(K, N, 512), kx2, x1, relR, BR.reshape(1, C // 2),
         W1r, b1r.reshape(1, C), W2r, b2r.reshape(1, BD),
         L2_Wkr,
         L2_W1, L2_b1.reshape(1, C), L2_W2, L2_b2.reshape(1, C)))

    return x2.reshape(N, G, C)


# speedup vs baseline: 19.0547x; 1.5306x over previous
import functools

import jax
import jax.numpy as jnp
from jax import lax
from jax.experimental import pallas as pl
from jax.experimental.pallas import tpu as pltpu
from jax.experimental.pallas import tpu_sc as plsc

N = 4096
G = 8
C = 64
BD = 64
K = 8
E = N * K

BRK = 128
BN = 128
TAU = 6.283185307179586


def _knn_body(pos_r, post_r, brow_r, bcol_r, out_r):
    p = pos_r[...]
    pt = post_r[...]
    sqr = jnp.sum(p * p, axis=1, keepdims=True)
    sqc = jnp.sum(pt * pt, axis=0, keepdims=True)
    cross = jnp.dot(p, pt, preferred_element_type=jnp.float32)
    d2 = sqr + sqc - 2.0 * cross
    rb = brow_r[...][:, 0:1]
    cb = bcol_r[...][0:1, :]
    d2 = jnp.where(rb != cb, jnp.inf, d2)
    iota = lax.broadcasted_iota(jnp.int32, d2.shape, 1)
    for k in range(K):
        rowmin = jnp.min(d2, axis=1, keepdims=True)
        cand = jnp.where(d2 <= rowmin, iota, N)
        idx = jnp.min(cand, axis=1, keepdims=True)
        out_r[:, k:k + 1] = idx
        d2 = jnp.where(iota == idx, jnp.inf, d2)


def _knn_call(posp8, post8, brow, bcol):
    return pl.pallas_call(
        _knn_body,
        grid=(N // BRK,),
        in_specs=[
            pl.BlockSpec((BRK, 8), lambda b: (b, 0)),
            pl.BlockSpec((8, N), lambda b: (0, 0)),
            pl.BlockSpec((BRK, 8), lambda b: (b, 0)),
            pl.BlockSpec((8, N), lambda b: (0, 0)),
        ],
        out_specs=pl.BlockSpec((BRK, K), lambda b: (b, 0)),
        out_shape=jax.ShapeDtypeStruct((N, K), jnp.int32),
    )(posp8, post8, brow, bcol)


_SC_CH = 128
_SC_NW = 32
_SC_ROWS = E // _SC_NW
_SC_NCH = _SC_ROWS // _SC_CH


def _gather_edges(table_x, table_p, idx):
    mesh = plsc.VectorSubcoreMesh(core_axis_name="c", subcore_axis_name="s")

    @functools.partial(
        pl.kernel,
        mesh=mesh,
        out_type=(
            jax.ShapeDtypeStruct((E, 512), jnp.float32),
            jax.ShapeDtypeStruct((E, 128), jnp.float32),
        ),
        scratch_types=[
            pltpu.VMEM((_SC_CH,), jnp.int32),
            pltpu.VMEM((_SC_CH, 512), jnp.float32),
            pltpu.VMEM((_SC_CH, 128), jnp.float32),
            pltpu.SemaphoreType.DMA,
            pltpu.SemaphoreType.DMA,
        ],
    )
    def gk(tx_hbm, tp_hbm, idx_hbm, ox_hbm, op_hbm, idx_v, rx_v, rp_v, s1, s2):
        wid = lax.axis_index("s") * 2 + lax.axis_index("c")
        base = wid * _SC_ROWS

        def chunk(i, carry):
            off = base + i * _SC_CH
            pltpu.sync_copy(idx_hbm.at[pl.ds(off, _SC_CH)], idx_v)
            a = pltpu.async_copy(tx_hbm.at[idx_v], rx_v, s1)
            b = pltpu.async_copy(tp_hbm.at[idx_v], rp_v, s2)
            a.wait()
            b.wait()
            pltpu.sync_copy(rx_v, ox_hbm.at[pl.ds(off, _SC_CH)])
            pltpu.sync_copy(rp_v, op_hbm.at[pl.ds(off, _SC_CH)])
            return carry

        lax.fori_loop(0, _SC_NCH, chunk, 0)

    return gk(table_x, table_p, idx)


def _gather_edges_x(table_x, idx):
    mesh = plsc.VectorSubcoreMesh(core_axis_name="c", subcore_axis_name="s")

    @functools.partial(
        pl.kernel,
        mesh=mesh,
        out_type=jax.ShapeDtypeStruct((E, 512), jnp.float32),
        scratch_types=[
            pltpu.VMEM((_SC_CH,), jnp.int32),
            pltpu.VMEM((_SC_CH, 512), jnp.float32),
            pltpu.SemaphoreType.DMA,
        ],
    )
    def gk(tx_hbm, idx_hbm, ox_hbm, idx_v, rx_v, s1):
        wid = lax.axis_index("s") * 2 + lax.axis_index("c")
        base = wid * _SC_ROWS

        def chunk(i, carry):
            off = base + i * _SC_CH
            pltpu.sync_copy(idx_hbm.at[pl.ds(off, _SC_CH)], idx_v)
            pltpu.async_copy(tx_hbm.at[idx_v], rx_v, s1).wait()
            pltpu.sync_copy(rx_v, ox_hbm.at[pl.ds(off, _SC_CH)])
            return carry

        lax.fori_loop(0, _SC_NCH, chunk, 0)

    return gk(table_x, idx)


_SS = (6.283185005187988, -41.3416633605957, 81.60163116455078,
       -76.56468200683594, 39.652915954589844)
_CC = (1.0, -19.739208221435547, 64.9393539428711, -85.45401000976562,
       60.15278244018555, -25.04315948486328)


def _sincos_2pi(q):
    n2 = jnp.round(2.0 * q)
    v = q - 0.5 * n2
    odd = lax.convert_element_type(n2, jnp.int32) << 31
    u = v * v
    s = _SS[4]
    for k in (3, 2, 1, 0):
        s = u * s + _SS[k]
    s = v * s
    c = _CC[5]
    for k in (4, 3, 2, 1, 0):
        c = u * c + _CC[k]
    sb = lax.bitcast_convert_type(s, jnp.int32) ^ odd
    cb = lax.bitcast_convert_type(c, jnp.int32) ^ odd
    return (lax.bitcast_convert_type(sb, jnp.float32),
            lax.bitcast_convert_type(cb, jnp.float32))


def _kR_mix(relR_r, br_r, w1r_r, b1r_r, w2r_r, b2r_r, wkr_r):
    pR = TAU * (relR_r[...] * br_r[...])
    featR = jnp.concatenate([jnp.sin(pR), jnp.cos(pR)], axis=1)
    hR = jax.nn.gelu(jnp.dot(featR, w1r_r[...],
                             preferred_element_type=jnp.float32) + b1r_r[...])
    kbR = jax.nn.gelu(jnp.dot(hR, w2r_r[...],
                              preferred_element_type=jnp.float32) + b2r_r[...])
    return jnp.dot(kbR, wkr_r[...], preferred_element_type=jnp.float32)


def _node_update(aggs, kR, x, w1_r, b1_r, w2_r, b2_r, out_r):
    for gi in range(G):
        xg = aggs[0] * kR[gi * G:gi * G + 1, :]
        for m in range(1, G):
            xg = xg + aggs[m] * kR[gi * G + m:gi * G + m + 1, :]
        xg = xg * 0.125
        h = jax.nn.gelu(jnp.dot(xg, w1_r[...],
                                preferred_element_type=jnp.float32) + b1_r[...])
        h = jnp.dot(h, w2_r[...], preferred_element_type=jnp.float32) + b2_r[...]
        out_r[:, gi * C:(gi + 1) * C] = x[:, gi * C:(gi + 1) * C] + h


def _layer1_body(g_r, gp_r, pos_r, x_r, gt_r, sel_r, relR_r, bxt0_r, bxt1_r,
                 br_r, w1bd_r, b1t_r, w2bd_r, b2t_r, w1r_r, b1r_r, w2r_r,
                 b2r_r, wkxbd_r, wkx2bd_r, wkr_r, w1_r, b1_r, w2_r, b2_r,
                 out_r, out2_r):
    Er = K * BN
    gp = gp_r[...].reshape(Er, 128)
    pos = pos_r[...]
    posrep = jnp.concatenate([pos] * K, axis=0)
    rel = gp - posrep
    rz = jnp.dot(rel, gt_r[...], preferred_element_type=jnp.float32)
    r2 = jnp.sum(rel * rel, axis=1, keepdims=True)
    rzsq = rz * rz

    bxt0 = bxt0_r[...]
    bxt1 = bxt1_r[...]
    sel = sel_r[...]
    b1t = b1t_r[...]
    b2t = b2t_r[...]

    g2 = g_r[...].reshape(Er, 512)
    x = x_r[...]

    kR = _kR_mix(relR_r, br_r, w1r_r, b1r_r, w2r_r, b2r_r, wkr_r)

    aggp = []
    for grp in range(2):
        selg = sel[:, 128 * grp:128 * (grp + 1)]
        rz_b = jnp.dot(rz, selg, preferred_element_type=jnp.float32)
        rzsq_b = jnp.dot(rzsq, selg, preferred_element_type=jnp.float32)
        rxy_b = jnp.sqrt(jnp.maximum(r2 - rzsq_b, 0.0) + 1e-12)
        q = rxy_b * bxt0 + rz_b * bxt1
        s_all, c_all = _sincos_2pi(q)
        feat = jnp.concatenate([s_all, c_all], axis=1)
        h = jax.nn.gelu(jnp.dot(feat, w1bd_r[...],
                                preferred_element_type=jnp.float32) + b1t)
        kb = jax.nn.gelu(jnp.dot(h, w2bd_r[...],
                                 preferred_element_type=jnp.float32) + b2t)
        kx = jnp.dot(kb, wkxbd_r[...], preferred_element_type=jnp.float32)
        kx2 = jnp.dot(kb, wkx2bd_r[...], preferred_element_type=jnp.float32)
        out2_r[:, :, 256 * grp:256 * (grp + 1)] = kx2.reshape(K, BN, 256)
        msg = g2[:, 256 * grp:256 * (grp + 1)] * kx
        agg = msg[0:BN]
        for k in range(1, K):
            agg = agg + msg[k * BN:(k + 1) * BN]
        aggp.append(agg * 0.125)

    aggs = [aggp[m // 4][:, (m % 4) * C:(m % 4 + 1) * C] for m in range(G)]
    _node_update(aggs, kR, x, w1_r, b1_r, w2_r, b2_r, out_r)


def _layer2_body(g_r, kx2_r, x_r, relR_r, br_r,
                 w1r_r, b1r_r, w2r_r, b2r_r, wkr_r,
                 w1_r, b1_r, w2_r, b2_r, out_r):
    Er = K * BN
    g2 = g_r[...].reshape(Er, 512)
    kxa = kx2_r[...].reshape(Er, 512)
    x = x_r[...]

    kR = _kR_mix(relR_r, br_r, w1r_r, b1r_r, w2r_r, b2r_r, wkr_r)

    msg = g2 * kxa
    acc = msg[0:BN]
    for k in range(1, K):
        acc = acc + msg[k * BN:(k + 1) * BN]
    acc = acc * 0.125
    aggs = [acc[:, m * C:(m + 1) * C] for m in range(G)]

    _node_update(aggs, kR, x, w1_r, b1_r, w2_r, b2_r, out_r)


def _layer1_specs():
    full = lambda shape: pl.BlockSpec(shape, lambda b: tuple(0 for _ in shape))
    in_specs = [
        pl.BlockSpec((K, BN, 512), lambda b: (0, b, 0)),
        pl.BlockSpec((K, BN, 128), lambda b: (0, b, 0)),
        pl.BlockSpec((BN, 128), lambda b: (b, 0)),
        pl.BlockSpec((BN, 512), lambda b: (b, 0)),
        full((128, G)), full((G, 256)), full((G * G, 1)),
        full((1, 128)), full((1, 128)), full((1, C // 2)),
        full((256, 256)), full((1, 256)), full((256, 256)), full((1, 256)),
        full((C, C)), full((1, C)), full((C, BD)), full((1, BD)),
        full((256, 256)), full((256, 256)), full((BD, C)),
        full((C, C)), full((1, C)), full((C, C)), full((1, C)),
    ]
    out_specs = [
        pl.BlockSpec((BN, 512), lambda b: (b, 0)),
        pl.BlockSpec((K, BN, 512), lambda b: (0, b, 0)),
    ]
    return in_specs, out_specs


def _layer1_call(args):
    in_specs, out_specs = _layer1_specs()
    return pl.pallas_call(
        _layer1_body,
        grid=(N // BN,),
        in_specs=in_specs,
        out_specs=out_specs,
        out_shape=[
            jax.ShapeDtypeStruct((N, 512), jnp.float32),
            jax.ShapeDtypeStruct((K, N, 512), jnp.float32),
        ],
    )(*args)


def _layer2_specs():
    full = lambda shape: pl.BlockSpec(shape, lambda b: tuple(0 for _ in shape))
    in_specs = [
        pl.BlockSpec((K, BN, 512), lambda b: (0, b, 0)),
        pl.BlockSpec((K, BN, 512), lambda b: (0, b, 0)),
        pl.BlockSpec((BN, 512), lambda b: (b, 0)),
        full((G * G, 1)), full((1, C // 2)),
        full((C, C)), full((1, C)), full((C, BD)), full((1, BD)),
        full((BD, C)),
        full((C, C)), full((1, C)), full((C, C)), full((1, C)),
    ]
    out_spec = pl.BlockSpec((BN, 512), lambda b: (b, 0))
    return in_specs, out_spec


def _layer2_call(args):
    in_specs, out_spec = _layer2_specs()
    return pl.pallas_call(
        _layer2_body,
        grid=(N // BN,),
        in_specs=in_specs,
        out_specs=out_spec,
        out_shape=jax.ShapeDtypeStruct((N, 512), jnp.float32),
    )(*args)


def kernel(x, pos_Rd, batch_Rd, grid, Bx, BR, W1x, b1x, W2x, b2x,
           W1r, b1r, W2r, b2r, L1_Wkx, L1_Wkr, L1_W1, L1_b1, L1_W2, L1_b2,
           L2_Wkx, L2_Wkr, L2_W1, L2_b1, L2_W2, L2_b2):
    x2d = x.reshape(N, G * C)
    posp8 = jnp.pad(pos_Rd, ((0, 0), (0, 5)))
    post8 = posp8.T
    posp128 = jnp.pad(pos_Rd, ((0, 0), (0, 125)))
    bi = batch_Rd.astype(jnp.int32)
    brow = jnp.broadcast_to(bi[:, None], (N, 8))
    bcol = jnp.broadcast_to(bi[None, :], (8, N))

    nbr = _knn_call(posp8, post8, brow, bcol)
    idx = nbr.T.reshape(E)

    gridT128 = jnp.pad(grid, ((0, 0), (0, 125))).T
    relR = (grid @ grid.T).reshape(G * G, 1)

    eye4 = jnp.eye(4, dtype=jnp.float32)
    sel = jnp.zeros((G, 256), jnp.float32)
    for j in range(G):
        sel = sel.at[j, 32 * j:32 * (j + 1)].set(1.0)
    bxt0 = jnp.tile(Bx[0:1, :], (1, 4))
    bxt1 = jnp.tile(Bx[1:2, :], (1, 4))
    w1bd = jnp.concatenate([jnp.kron(eye4, W1x[:C // 2, :]),
                            jnp.kron(eye4, W1x[C // 2:, :])], axis=0)
    w2bd = jnp.kron(eye4, W2x)
    wkxbd = jnp.kron(eye4, L1_Wkx)
    wkx2bd = jnp.kron(eye4, L2_Wkx)
    b1t = jnp.tile(b1x.reshape(1, C), (1, 4))
    b2t = jnp.tile(b2x.reshape(1, BD), (1, 4))

    g1, gp1 = _gather_edges(x2d, posp128, idx)
    g13 = g1.reshape(K, N, 512)
    gp13 = gp1.reshape(K, N, 128)

    x1, kx2 = _layer1_call(
        (g13, gp13, posp128, x2d, gridT128, sel, relR,
         bxt0, bxt1, BR.reshape(1, C // 2),
         w1bd, b1t, w2bd, b2t,
         W1r, b1r.reshape(1, C), W2r, b2r.reshape(1, BD),
         wkxbd, wkx2bd, L1_Wkr,
         L1_W1, L1_b1.reshape(1, C), L1_W2, L1_b2.reshape(1, C)))

    g2 = _gather_edges_x(x1, idx)
    x2 = _layer2_call(
        (g2.reshape(K, N, 512), kx2, x1, relR, BR.reshape(1, C // 2),
         W1r, b1r.reshape(1, C), W2r, b2r.reshape(1, BD),
         L2_Wkr,
         L2_W1, L2_b1.reshape(1, C), L2_W2, L2_b2.reshape(1, C)))

    return x2.reshape(N, G, C)


# speedup vs baseline: 19.3382x; 1.0149x over previous
import functools

import jax
import jax.numpy as jnp
from jax import lax
from jax.experimental import pallas as pl
from jax.experimental.pallas import tpu as pltpu
from jax.experimental.pallas import tpu_sc as plsc

N = 4096
G = 8
C = 64
BD = 64
K = 8
E = N * K

BRK = 128
BN = 128
TAU = 6.283185307179586


def _knn_body(pos_r, post_r, brow_r, bcol_r, out_r):
    p = pos_r[...]
    pt = post_r[...]
    sqr = jnp.sum(p * p, axis=1, keepdims=True)
    sqc = jnp.sum(pt * pt, axis=0, keepdims=True)
    cross = jnp.dot(p, pt, preferred_element_type=jnp.float32)
    d2 = sqr + sqc - 2.0 * cross
    rb = brow_r[...][:, 0:1]
    cb = bcol_r[...][0:1, :]
    d2 = jnp.where(rb != cb, jnp.inf, d2)
    iota = lax.broadcasted_iota(jnp.int32, d2.shape, 1)
    for k in range(K):
        rowmin = jnp.min(d2, axis=1, keepdims=True)
        cand = jnp.where(d2 <= rowmin, iota, N)
        idx = jnp.min(cand, axis=1, keepdims=True)
        out_r[:, k:k + 1] = idx
        d2 = jnp.where(iota == idx, jnp.inf, d2)


def _knn_call(posp8, post8, brow, bcol):
    return pl.pallas_call(
        _knn_body,
        grid=(N // BRK,),
        in_specs=[
            pl.BlockSpec((BRK, 8), lambda b: (b, 0)),
            pl.BlockSpec((8, N), lambda b: (0, 0)),
            pl.BlockSpec((BRK, 8), lambda b: (b, 0)),
            pl.BlockSpec((8, N), lambda b: (0, 0)),
        ],
        out_specs=pl.BlockSpec((BRK, K), lambda b: (b, 0)),
        out_shape=jax.ShapeDtypeStruct((N, K), jnp.int32),
    )(posp8, post8, brow, bcol)


_SC_CH = 64
_SC_NW = 32
_SC_ROWS = E // _SC_NW
_SC_NCH = _SC_ROWS // _SC_CH


def _gather_edges(table_x, table_p, idx):
    mesh = plsc.VectorSubcoreMesh(core_axis_name="c", subcore_axis_name="s")

    @functools.partial(
        pl.kernel,
        mesh=mesh,
        out_type=(
            jax.ShapeDtypeStruct((E, 512), jnp.float32),
            jax.ShapeDtypeStruct((E, 128), jnp.float32),
        ),
        scratch_types=[
            pltpu.VMEM((_SC_NCH, _SC_CH), jnp.int32),
            pltpu.VMEM((2, _SC_CH, 512), jnp.float32),
            pltpu.VMEM((2, _SC_CH, 128), jnp.float32),
            pltpu.SemaphoreType.DMA,
            pltpu.SemaphoreType.DMA,
            pltpu.SemaphoreType.DMA,
            pltpu.SemaphoreType.DMA,
        ],
    )
    def gk(tx_hbm, tp_hbm, idx_hbm, ox_hbm, op_hbm, idx_v, rx_v, rp_v,
           sx0, sx1, sp0, sp1):
        wid = lax.axis_index("s") * 2 + lax.axis_index("c")
        base = wid * _SC_ROWS
        pltpu.sync_copy(idx_hbm.at[wid], idx_v)
        sx = (sx0, sx1)
        sp = (sp0, sp1)
        cps = []
        for i in range(_SC_NCH):
            sl = i % 2
            cx = pltpu.async_copy(tx_hbm.at[idx_v.at[i]], rx_v.at[sl], sx[sl])
            cp = pltpu.async_copy(tp_hbm.at[idx_v.at[i]], rp_v.at[sl], sp[sl])
            cps.append((cx, cp))
            if i >= 1:
                pcx, pcp = cps[i - 1]
                pcx.wait()
                pcp.wait()
                off = base + (i - 1) * _SC_CH
                pltpu.sync_copy(rx_v.at[1 - sl], ox_hbm.at[pl.ds(off, _SC_CH)])
                pltpu.sync_copy(rp_v.at[1 - sl], op_hbm.at[pl.ds(off, _SC_CH)])
        cx, cp = cps[-1]
        cx.wait()
        cp.wait()
        last = _SC_NCH - 1
        off = base + last * _SC_CH
        pltpu.sync_copy(rx_v.at[last % 2], ox_hbm.at[pl.ds(off, _SC_CH)])
        pltpu.sync_copy(rp_v.at[last % 2], op_hbm.at[pl.ds(off, _SC_CH)])

    return gk(table_x, table_p, idx.reshape(_SC_NW, _SC_NCH, _SC_CH))


def _gather_edges_x(table_x, idx):
    mesh = plsc.VectorSubcoreMesh(core_axis_name="c", subcore_axis_name="s")

    @functools.partial(
        pl.kernel,
        mesh=mesh,
        out_type=jax.ShapeDtypeStruct((E, 512), jnp.float32),
        scratch_types=[
            pltpu.VMEM((_SC_NCH, _SC_CH), jnp.int32),
            pltpu.VMEM((2, _SC_CH, 512), jnp.float32),
            pltpu.SemaphoreType.DMA,
            pltpu.SemaphoreType.DMA,
        ],
    )
    def gk(tx_hbm, idx_hbm, ox_hbm, idx_v, rx_v, sx0, sx1):
        wid = lax.axis_index("s") * 2 + lax.axis_index("c")
        base = wid * _SC_ROWS
        pltpu.sync_copy(idx_hbm.at[wid], idx_v)
        sx = (sx0, sx1)
        cps = []
        for i in range(_SC_NCH):
            sl = i % 2
            cps.append(pltpu.async_copy(tx_hbm.at[idx_v.at[i]], rx_v.at[sl],
                                        sx[sl]))
            if i >= 1:
                cps[i - 1].wait()
                off = base + (i - 1) * _SC_CH
                pltpu.sync_copy(rx_v.at[1 - sl], ox_hbm.at[pl.ds(off, _SC_CH)])
        cps[-1].wait()
        last = _SC_NCH - 1
        off = base + last * _SC_CH
        pltpu.sync_copy(rx_v.at[last % 2], ox_hbm.at[pl.ds(off, _SC_CH)])

    return gk(table_x, idx.reshape(_SC_NW, _SC_NCH, _SC_CH))


_SS = (6.283185005187988, -41.3416633605957, 81.60163116455078,
       -76.56468200683594, 39.652915954589844)
_CC = (1.0, -19.739208221435547, 64.9393539428711, -85.45401000976562,
       60.15278244018555, -25.04315948486328)


def _sincos_2pi(q):
    n2 = jnp.round(2.0 * q)
    v = q - 0.5 * n2
    odd = lax.convert_element_type(n2, jnp.int32) << 31
    u = v * v
    s = _SS[4]
    for k in (3, 2, 1, 0):
        s = u * s + _SS[k]
    s = v * s
    c = _CC[5]
    for k in (4, 3, 2, 1, 0):
        c = u * c + _CC[k]
    sb = lax.bitcast_convert_type(s, jnp.int32) ^ odd
    cb = lax.bitcast_convert_type(c, jnp.int32) ^ odd
    return (lax.bitcast_convert_type(sb, jnp.float32),
            lax.bitcast_convert_type(cb, jnp.float32))


def _kR_mix(relR_r, br_r, w1r_r, b1r_r, w2r_r, b2r_r, wkr_r):
    pR = TAU * (relR_r[...] * br_r[...])
    featR = jnp.concatenate([jnp.sin(pR), jnp.cos(pR)], axis=1)
    hR = jax.nn.gelu(jnp.dot(featR, w1r_r[...],
                             preferred_element_type=jnp.float32) + b1r_r[...])
    kbR = jax.nn.gelu(jnp.dot(hR, w2r_r[...],
                              preferred_element_type=jnp.float32) + b2r_r[...])
    return jnp.dot(kbR, wkr_r[...], preferred_element_type=jnp.float32)


def _node_update(aggs, kR, x, w1_r, b1_r, w2_r, b2_r, out_r):
    for gi in range(G):
        xg = aggs[0] * kR[gi * G:gi * G + 1, :]
        for m in range(1, G):
            xg = xg + aggs[m] * kR[gi * G + m:gi * G + m + 1, :]
        xg = xg * 0.125
        h = jax.nn.gelu(jnp.dot(xg, w1_r[...],
                                preferred_element_type=jnp.float32) + b1_r[...])
        h = jnp.dot(h, w2_r[...], preferred_element_type=jnp.float32) + b2_r[...]
        out_r[:, gi * C:(gi + 1) * C] = x[:, gi * C:(gi + 1) * C] + h


def _layer1_body(g_r, gp_r, pos_r, x_r, gt_r, sel_r, relR_r, bxt0_r, bxt1_r,
                 br_r, w1bd_r, b1t_r, w2bd_r, b2t_r, w1r_r, b1r_r, w2r_r,
                 b2r_r, wkxbd_r, wkx2bd_r, wkr_r, w1_r, b1_r, w2_r, b2_r,
                 out_r, out2_r):
    Er = K * BN
    gp = gp_r[...].reshape(Er, 128)
    pos = pos_r[...]
    posrep = jnp.concatenate([pos] * K, axis=0)
    rel = gp - posrep
    rz = jnp.dot(rel, gt_r[...], preferred_element_type=jnp.float32)
    r2 = jnp.sum(rel * rel, axis=1, keepdims=True)
    rzsq = rz * rz

    bxt0 = bxt0_r[...]
    bxt1 = bxt1_r[...]
    sel = sel_r[...]
    b1t = b1t_r[...]
    b2t = b2t_r[...]

    g2 = g_r[...].reshape(Er, 512)
    x = x_r[...]

    kR = _kR_mix(relR_r, br_r, w1r_r, b1r_r, w2r_r, b2r_r, wkr_r)

    aggp = []
    for grp in range(2):
        selg = sel[:, 128 * grp:128 * (grp + 1)]
        rz_b = jnp.dot(rz, selg, preferred_element_type=jnp.float32)
        rzsq_b = jnp.dot(rzsq, selg, preferred_element_type=jnp.float32)
        rxy_b = jnp.sqrt(jnp.maximum(r2 - rzsq_b, 0.0) + 1e-12)
        q = rxy_b * bxt0 + rz_b * bxt1
        s_all, c_all = _sincos_2pi(q)
        feat = jnp.concatenate([s_all, c_all], axis=1)
        h = jax.nn.gelu(jnp.dot(feat, w1bd_r[...],
                                preferred_element_type=jnp.float32) + b1t)
        kb = jax.nn.gelu(jnp.dot(h, w2bd_r[...],
                                 preferred_element_type=jnp.float32) + b2t)
        kx = jnp.dot(kb, wkxbd_r[...], preferred_element_type=jnp.float32)
        kx2 = jnp.dot(kb, wkx2bd_r[...], preferred_element_type=jnp.float32)
        out2_r[:, :, 256 * grp:256 * (grp + 1)] = kx2.reshape(K, BN, 256)
        msg = g2[:, 256 * grp:256 * (grp + 1)] * kx
        agg = msg[0:BN]
        for k in range(1, K):
            agg = agg + msg[k * BN:(k + 1) * BN]
        aggp.append(agg * 0.125)

    aggs = [aggp[m // 4][:, (m % 4) * C:(m % 4 + 1) * C] for m in range(G)]
    _node_update(aggs, kR, x, w1_r, b1_r, w2_r, b2_r, out_r)


def _layer2_body(g_r, kx2_r, x_r, relR_r, br_r,
                 w1r_r, b1r_r, w2r_r, b2r_r, wkr_r,
                 w1_r, b1_r, w2_r, b2_r, out_r):
    Er = K * BN
    g2 = g_r[...].reshape(Er, 512)
    kxa = kx2_r[...].reshape(Er, 512)
    x = x_r[...]

    kR = _kR_mix(relR_r, br_r, w1r_r, b1r_r, w2r_r, b2r_r, wkr_r)

    msg = g2 * kxa
    acc = msg[0:BN]
    for k in range(1, K):
        acc = acc + msg[k * BN:(k + 1) * BN]
    acc = acc * 0.125
    aggs = [acc[:, m * C:(m + 1) * C] for m in range(G)]

    _node_update(aggs, kR, x, w1_r, b1_r, w2_r, b2_r, out_r)


def _layer1_specs():
    full = lambda shape: pl.BlockSpec(shape, lambda b: tuple(0 for _ in shape))
    in_specs = [
        pl.BlockSpec((K, BN, 512), lambda b: (0, b, 0)),
        pl.BlockSpec((K, BN, 128), lambda b: (0, b, 0)),
        pl.BlockSpec((BN, 128), lambda b: (b, 0)),
        pl.BlockSpec((BN, 512), lambda b: (b, 0)),
        full((128, G)), full((G, 256)), full((G * G, 1)),
        full((1, 128)), full((1, 128)), full((1, C // 2)),
        full((256, 256)), full((1, 256)), full((256, 256)), full((1, 256)),
        full((C, C)), full((1, C)), full((C, BD)), full((1, BD)),
        full((256, 256)), full((256, 256)), full((BD, C)),
        full((C, C)), full((1, C)), full((C, C)), full((1, C)),
    ]
    out_specs = [
        pl.BlockSpec((BN, 512), lambda b: (b, 0)),
        pl.BlockSpec((K, BN, 512), lambda b: (0, b, 0)),
    ]
    return in_specs, out_specs


def _layer1_call(args):
    in_specs, out_specs = _layer1_specs()
    return pl.pallas_call(
        _layer1_body,
        grid=(N // BN,),
        in_specs=in_specs,
        out_specs=out_specs,
        out_shape=[
            jax.ShapeDtypeStruct((N, 512), jnp.float32),
            jax.ShapeDtypeStruct((K, N, 512), jnp.float32),
        ],
    )(*args)


def _layer2_specs():
    full = lambda shape: pl.BlockSpec(shape, lambda b: tuple(0 for _ in shape))
    in_specs = [
        pl.BlockSpec((K, BN, 512), lambda b: (0, b, 0)),
        pl.BlockSpec((K, BN, 512), lambda b: (0, b, 0)),
        pl.BlockSpec((BN, 512), lambda b: (b, 0)),
        full((G * G, 1)), full((1, C // 2)),
        full((C, C)), full((1, C)), full((C, BD)), full((1, BD)),
        full((BD, C)),
        full((C, C)), full((1, C)), full((C, C)), full((1, C)),
    ]
    out_spec = pl.BlockSpec((BN, 512), lambda b: (b, 0))
    return in_specs, out_spec


def _layer2_call(args):
    in_specs, out_spec = _layer2_specs()
    return pl.pallas_call(
        _layer2_body,
        grid=(N // BN,),
        in_specs=in_specs,
        out_specs=out_spec,
        out_shape=jax.ShapeDtypeStruct((N, 512), jnp.float32),
    )(*args)


def kernel(x, pos_Rd, batch_Rd, grid, Bx, BR, W1x, b1x, W2x, b2x,
           W1r, b1r, W2r, b2r, L1_Wkx, L1_Wkr, L1_W1, L1_b1, L1_W2, L1_b2,
           L2_Wkx, L2_Wkr, L2_W1, L2_b1, L2_W2, L2_b2):
    x2d = x.reshape(N, G * C)
    posp8 = jnp.pad(pos_Rd, ((0, 0), (0, 5)))
    post8 = posp8.T
    posp128 = jnp.pad(pos_Rd, ((0, 0), (0, 125)))
    bi = batch_Rd.astype(jnp.int32)
    brow = jnp.broadcast_to(bi[:, None], (N, 8))
    bcol = jnp.broadcast_to(bi[None, :], (8, N))

    nbr = _knn_call(posp8, post8, brow, bcol)
    idx = nbr.T.reshape(E)

    gridT128 = jnp.pad(grid, ((0, 0), (0, 125))).T
    relR = (grid @ grid.T).reshape(G * G, 1)

    eye4 = jnp.eye(4, dtype=jnp.float32)
    sel = jnp.zeros((G, 256), jnp.float32)
    for j in range(G):
        sel = sel.at[j, 32 * j:32 * (j + 1)].set(1.0)
    bxt0 = jnp.tile(Bx[0:1, :], (1, 4))
    bxt1 = jnp.tile(Bx[1:2, :], (1, 4))
    w1bd = jnp.concatenate([jnp.kron(eye4, W1x[:C // 2, :]),
                            jnp.kron(eye4, W1x[C // 2:, :])], axis=0)
    w2bd = jnp.kron(eye4, W2x)
    wkxbd = jnp.kron(eye4, L1_Wkx)
    wkx2bd = jnp.kron(eye4, L2_Wkx)
    b1t = jnp.tile(b1x.reshape(1, C), (1, 4))
    b2t = jnp.tile(b2x.reshape(1, BD), (1, 4))

    g1, gp1 = _gather_edges(x2d, posp128, idx)
    g13 = g1.reshape(K, N, 512)
    gp13 = gp1.reshape(K, N, 128)

    x1, kx2 = _layer1_call(
        (g13, gp13, posp128, x2d, gridT128, sel, relR,
         bxt0, bxt1, BR.reshape(1, C // 2),
         w1bd, b1t, w2bd, b2t,
         W1r, b1r.reshape(1, C), W2r, b2r.reshape(1, BD),
         wkxbd, wkx2bd, L1_Wkr,
         L1_W1, L1_b1.reshape(1, C), L1_W2, L1_b2.reshape(1, C)))

    g2 = _gather_edges_x(x1, idx)
    x2 = _layer2_call(
        (g2.reshape(K, N, 512), kx2, x1, relR, BR.reshape(1, C // 2),
         W1r, b1r.reshape(1, C), W2r, b2r.reshape(1, BD),
         L2_Wkr,
         L2_W1, L2_b1.reshape(1, C), L2_W2, L2_b2.reshape(1, C)))

    return x2.reshape(N, G, C)
